# baseline (speedup 1.0000x reference)
import jax
import jax.numpy as jnp
from jax.experimental import pallas as pl
from jax.experimental.pallas import tpu as pltpu

N_EMB = 1024
E_DIM = 64
COMMIT_COST = 0.25
B, H, W = 32, 32, 32
N_TOK = B * H * W
ROWS = H * W
GRID = B


def _vq_body(z_ref, emb_ref, enc_ref, zq_ref, idx_ref, loss_ref, ppl_ref,
             sum_ref, cnt_ref):
    i = pl.program_id(0)

    @pl.when(i == 0)
    def _init():
        sum_ref[...] = jnp.zeros_like(sum_ref)
        cnt_ref[...] = jnp.zeros_like(cnt_ref)

    z = jnp.transpose(z_ref[0].reshape(E_DIM, ROWS), (1, 0))
    emb = emb_ref[...]

    z2 = jnp.sum(z * z, axis=1, keepdims=True)
    e2 = jnp.sum(emb * emb, axis=1, keepdims=True)
    mm2 = jax.lax.dot_general(z, emb + emb, (((1,), (1,)), ((), ())),
                              preferred_element_type=jnp.float32)
    d = (z2 + e2[:, 0][None, :]) - mm2

    dmin = jnp.min(d, axis=1, keepdims=True)
    colsf = jax.lax.broadcasted_iota(jnp.int32, (ROWS, N_EMB), 1).astype(jnp.float32)
    idxf = jnp.min(jnp.where(d == dmin, colsf, float(N_EMB)), axis=1,
                   keepdims=True)

    oh = jnp.where(colsf == idxf, 1.0, 0.0).astype(jnp.float32)
    enc_ref[...] = oh
    zq = jax.lax.dot_general(oh, emb, (((1,), (0,)), ((), ())),
                             preferred_element_type=jnp.float32)
    zq_ref[0] = jnp.transpose(zq.reshape(H, W, E_DIM), (1, 2, 0))
    idx_ref[...] = idxf.astype(jnp.int32)

    diff = zq - z
    sum_ref[...] += jnp.sum(diff * diff, axis=(0, 1), keepdims=True)
    ones_row = jnp.full((1, ROWS), 1.0, jnp.float32)
    cnt_ref[...] += jax.lax.dot_general(ones_row, oh, (((1,), (0,)), ((), ())),
                                        preferred_element_type=jnp.float32)

    @pl.when(i == GRID - 1)
    def _finish():
        mse = sum_ref[0, 0] / (N_TOK * E_DIM)
        loss_ref[...] = jnp.full((1, 1), 0.0, jnp.float32) + mse * (1.0 + COMMIT_COST)
        e_mean = cnt_ref[...] / N_TOK
        ent = -jnp.sum(e_mean * jnp.log(e_mean + 1e-10), axis=(0, 1), keepdims=True)
        ppl_ref[...] = jnp.exp(ent)


def _vq_call(z, emb):
    return pl.pallas_call(
        _vq_body,
        grid=(GRID,),
        in_specs=[
            pl.BlockSpec((1, E_DIM, H, W), lambda i: (i, 0, 0, 0)),
            pl.BlockSpec((N_EMB, E_DIM), lambda i: (0, 0)),
        ],
        out_specs=[
            pl.BlockSpec((ROWS, N_EMB), lambda i: (i, 0)),
            pl.BlockSpec((1, W, E_DIM, H), lambda i: (i, 0, 0, 0)),
            pl.BlockSpec((ROWS, 1), lambda i: (i, 0)),
            pl.BlockSpec((1, 1), lambda i: (0, 0)),
            pl.BlockSpec((1, 1), lambda i: (0, 0)),
        ],
        out_shape=[
            jax.ShapeDtypeStruct((N_TOK, N_EMB), jnp.float32),
            jax.ShapeDtypeStruct((B, W, E_DIM, H), jnp.float32),
            jax.ShapeDtypeStruct((N_TOK, 1), jnp.int32),
            jax.ShapeDtypeStruct((1, 1), jnp.float32),
            jax.ShapeDtypeStruct((1, 1), jnp.float32),
        ],
        scratch_shapes=[
            pltpu.VMEM((1, 1), jnp.float32),
            pltpu.VMEM((1, N_EMB), jnp.float32),
        ],
        compiler_params=pltpu.CompilerParams(
            dimension_semantics=("arbitrary",),
        ),
    )(z, emb)


def kernel(z, emb):
    enc, z_q, idx, loss, ppl = _vq_call(z, emb)
    return (loss[0, 0], z_q, ppl[0, 0], enc, idx)


# speedup vs baseline: 1.4787x; 1.4787x over previous
import jax
import jax.numpy as jnp
from jax.experimental import pallas as pl
from jax.experimental.pallas import tpu as pltpu

N_EMB = 1024
E_DIM = 64
COMMIT_COST = 0.25
N_TOK = 32768
ROWS = 1024
GRID = N_TOK // ROWS


def _vq_body(z_ref, emb_ref, enc_ref, zq_ref, idx_ref, loss_ref, ppl_ref,
             sum_ref, cnt_ref):
    i = pl.program_id(0)

    @pl.when(i == 0)
    def _init():
        sum_ref[...] = jnp.zeros_like(sum_ref)
        cnt_ref[...] = jnp.zeros_like(cnt_ref)

    z = z_ref[...]
    emb = emb_ref[...]

    z2 = jnp.sum(z * z, axis=1, keepdims=True)
    e2 = jnp.sum(emb * emb, axis=1, keepdims=True)
    mm2 = jax.lax.dot_general(z, emb + emb, (((1,), (1,)), ((), ())),
                              preferred_element_type=jnp.float32)
    d = (z2 + e2[:, 0][None, :]) - mm2

    dmin = jnp.min(d, axis=1, keepdims=True)
    colsf = jax.lax.broadcasted_iota(jnp.int32, (ROWS, N_EMB), 1).astype(jnp.float32)
    idxf = jnp.min(jnp.where(d == dmin, colsf, float(N_EMB)), axis=1,
                   keepdims=True)

    oh = jnp.where(colsf == idxf, 1.0, 0.0).astype(jnp.float32)
    enc_ref[...] = oh
    zq = jax.lax.dot_general(oh, emb, (((1,), (0,)), ((), ())),
                             preferred_element_type=jnp.float32)
    zq_ref[...] = zq
    idx_ref[...] = idxf.astype(jnp.int32)

    diff = zq - z
    sum_ref[...] += jnp.sum(diff * diff, axis=(0, 1), keepdims=True)
    ones_row = jnp.full((1, ROWS), 1.0, jnp.float32)
    cnt_ref[...] += jax.lax.dot_general(ones_row, oh, (((1,), (0,)), ((), ())),
                                        preferred_element_type=jnp.float32)

    @pl.when(i == GRID - 1)
    def _finish():
        mse = sum_ref[0, 0] / (N_TOK * E_DIM)
        loss_ref[...] = jnp.full((1, 1), 0.0, jnp.float32) + mse * (1.0 + COMMIT_COST)
        e_mean = cnt_ref[...] / N_TOK
        ent = -jnp.sum(e_mean * jnp.log(e_mean + 1e-10), axis=(0, 1), keepdims=True)
        ppl_ref[...] = jnp.exp(ent)


def _vq_call(z_flat, emb):
    return pl.pallas_call(
        _vq_body,
        grid=(GRID,),
        in_specs=[
            pl.BlockSpec((ROWS, E_DIM), lambda i: (i, 0)),
            pl.BlockSpec((N_EMB, E_DIM), lambda i: (0, 0)),
        ],
        out_specs=[
            pl.BlockSpec((ROWS, N_EMB), lambda i: (i, 0)),
            pl.BlockSpec((ROWS, E_DIM), lambda i: (i, 0)),
            pl.BlockSpec((ROWS, 1), lambda i: (i, 0)),
            pl.BlockSpec((1, 1), lambda i: (0, 0)),
            pl.BlockSpec((1, 1), lambda i: (0, 0)),
        ],
        out_shape=[
            jax.ShapeDtypeStruct((N_TOK, N_EMB), jnp.float32),
            jax.ShapeDtypeStruct((N_TOK, E_DIM), jnp.float32),
            jax.ShapeDtypeStruct((N_TOK, 1), jnp.int32),
            jax.ShapeDtypeStruct((1, 1), jnp.float32),
            jax.ShapeDtypeStruct((1, 1), jnp.float32),
        ],
        scratch_shapes=[
            pltpu.VMEM((1, 1), jnp.float32),
            pltpu.VMEM((1, N_EMB), jnp.float32),
        ],
        compiler_params=pltpu.CompilerParams(
            dimension_semantics=("arbitrary",),
        ),
    )(z_flat, emb)


def kernel(z, emb):
    z_p = jnp.transpose(z, (0, 2, 3, 1))
    z_flat = z_p.reshape(-1, E_DIM)
    enc, zq_flat, idx, loss, ppl = _vq_call(z_flat, emb)
    z_q = jnp.transpose(zq_flat.reshape(z_p.shape), (0, 2, 3, 1))
    return (loss[0, 0], z_q, ppl[0, 0], enc, idx)


# speedup vs baseline: 1.4794x; 1.0005x over previous
import jax
import jax.numpy as jnp
from jax.experimental import pallas as pl
from jax.experimental.pallas import tpu as pltpu

N_EMB = 1024
E_DIM = 64
COMMIT_COST = 0.25
N_TOK = 32768
ROWS = 1024
GRID = N_TOK // ROWS


def _vq_body(z_ref, emb_ref, enc_ref, zq_ref, idx_ref, psum_ref, pcnt_ref):
    z = z_ref[...]
    emb = emb_ref[...]

    z2 = jnp.sum(z * z, axis=1, keepdims=True)
    e2 = jnp.sum(emb * emb, axis=1, keepdims=True)
    mm2 = jax.lax.dot_general(z, emb + emb, (((1,), (1,)), ((), ())),
                              preferred_element_type=jnp.float32)
    d = (z2 + e2[:, 0][None, :]) - mm2

    dmin = jnp.min(d, axis=1, keepdims=True)
    colsf = jax.lax.broadcasted_iota(jnp.int32, (ROWS, N_EMB), 1).astype(jnp.float32)
    idxf = jnp.min(jnp.where(d == dmin, colsf, float(N_EMB)), axis=1,
                   keepdims=True)

    oh = jnp.where(colsf == idxf, 1.0, 0.0).astype(jnp.float32)
    enc_ref[...] = oh
    zq = jax.lax.dot_general(oh, emb, (((1,), (0,)), ((), ())),
                             preferred_element_type=jnp.float32)
    zq_ref[...] = zq
    idx_ref[...] = idxf.astype(jnp.int32)

    diff = zq - z
    psum_ref[...] = jnp.sum(diff * diff, axis=(0, 1), keepdims=True)[:, :, None]
    ones_row = jnp.full((1, ROWS), 1.0, jnp.float32)
    pcnt_ref[...] = jax.lax.dot_general(ones_row, oh, (((1,), (0,)), ((), ())),
                                        preferred_element_type=jnp.float32)[None]


def _finish_body(psum_ref, pcnt_ref, loss_ref, ppl_ref):
    total = jnp.sum(psum_ref[...], axis=(0, 1, 2))
    mse = total / (N_TOK * E_DIM)
    loss_ref[...] = jnp.full((1, 1), 0.0, jnp.float32) + mse * (1.0 + COMMIT_COST)
    e_mean = jnp.sum(pcnt_ref[...], axis=0) / N_TOK
    ent = -jnp.sum(e_mean * jnp.log(e_mean + 1e-10), axis=(0, 1), keepdims=True)
    ppl_ref[...] = jnp.exp(ent)


def _vq_call(z_flat, emb):
    enc, zq, idx, psum, pcnt = pl.pallas_call(
        _vq_body,
        grid=(GRID,),
        in_specs=[
            pl.BlockSpec((ROWS, E_DIM), lambda i: (i, 0)),
            pl.BlockSpec((N_EMB, E_DIM), lambda i: (0, 0)),
        ],
        out_specs=[
            pl.BlockSpec((ROWS, N_EMB), lambda i: (i, 0)),
            pl.BlockSpec((ROWS, E_DIM), lambda i: (i, 0)),
            pl.BlockSpec((ROWS, 1), lambda i: (i, 0)),
            pl.BlockSpec((1, 1, 1), lambda i: (i, 0, 0)),
            pl.BlockSpec((1, 1, N_EMB), lambda i: (i, 0, 0)),
        ],
        out_shape=[
            jax.ShapeDtypeStruct((N_TOK, N_EMB), jnp.float32),
            jax.ShapeDtypeStruct((N_TOK, E_DIM), jnp.float32),
            jax.ShapeDtypeStruct((N_TOK, 1), jnp.int32),
            jax.ShapeDtypeStruct((GRID, 1, 1), jnp.float32),
            jax.ShapeDtypeStruct((GRID, 1, N_EMB), jnp.float32),
        ],
        compiler_params=pltpu.CompilerParams(
            dimension_semantics=("parallel",),
        ),
    )(z_flat, emb)
    loss, ppl = pl.pallas_call(
        _finish_body,
        out_shape=[
            jax.ShapeDtypeStruct((1, 1), jnp.float32),
            jax.ShapeDtypeStruct((1, 1), jnp.float32),
        ],
    )(psum, pcnt)
    return enc, zq, idx, loss, ppl


def kernel(z, emb):
    z_p = jnp.transpose(z, (0, 2, 3, 1))
    z_flat = z_p.reshape(-1, E_DIM)
    enc, zq_flat, idx, loss, ppl = _vq_call(z_flat, emb)
    z_q = jnp.transpose(zq_flat.reshape(z_p.shape), (0, 2, 3, 1))
    return (loss[0, 0], z_q, ppl[0, 0], enc, idx)
